# Initial kernel scaffold; baseline (speedup 1.0000x reference)
import jax
import jax.numpy as jnp
from jax.experimental import pallas as pl


def kernel(coords, features):
    raise NotImplementedError("write your pallas kernel here")


# speedup vs baseline: 1.9421x; 1.9421x over previous
import functools

import jax
import jax.numpy as jnp
from jax import lax
from jax.experimental import pallas as pl
from jax.experimental.pallas import tpu as pltpu
from jax.experimental.pallas import tpu_sc as plsc

D = H = W = 128
C = 16
N = 524288
V = D * H * W

NC = 2
NS = 16
NW = NC * NS
NPW = N // NW
B = 256
NBLK = NPW // B
G = B // 16
NIDX = 8 * B
ILEN = 128
NSTREAM = NIDX // ILEN

_mesh = plsc.VectorSubcoreMesh(core_axis_name="c", subcore_axis_name="s")


@functools.partial(
    pl.kernel,
    mesh=_mesh,
    compiler_params=pltpu.CompilerParams(use_tc_tiling_on_sc=False),
    out_type=jax.ShapeDtypeStruct((N, C), jnp.float32),
    scratch_types=[
        pltpu.VMEM((B,), jnp.float32),
        pltpu.VMEM((B,), jnp.float32),
        pltpu.VMEM((B,), jnp.float32),
        pltpu.VMEM((NIDX,), jnp.int32),
        pltpu.VMEM((NIDX,), jnp.float32),
        pltpu.VMEM((NIDX, C), jnp.float32),
        pltpu.VMEM((B, C), jnp.float32),
        pltpu.SemaphoreType.DMA,
    ],
)
def _sample_sc(xs_hbm, ys_hbm, zs_hbm, table_hbm, out_hbm,
               xv, yv, zv, idx_v, w_v, rows_v, out_v, sem):
    wid = lax.axis_index("s") * NC + lax.axis_index("c")
    lanes = lax.iota(jnp.int32, 16)

    def blk_body(blk, carry):
        base = wid * NPW + blk * B
        pltpu.sync_copy(xs_hbm.at[pl.ds(base, B)], xv)
        pltpu.sync_copy(ys_hbm.at[pl.ds(base, B)], yv)
        pltpu.sync_copy(zs_hbm.at[pl.ds(base, B)], zv)

        def grp_body(g, c2):
            b0 = g * 16
            cx = xv[pl.ds(b0, 16)]
            cy = yv[pl.ds(b0, 16)]
            cz = zv[pl.ds(b0, 16)]

            def axis(cu, ext):
                gg = cu * 2.0 - 1.0
                u = (gg + 1.0) * 0.5 * (ext - 1)
                u = jnp.minimum(jnp.maximum(u, 0.0), float(ext - 1))
                u0 = u.astype(jnp.int32)
                wu = u - u0.astype(jnp.float32)
                u1 = jnp.minimum(u0 + 1, ext - 1)
                return u0, u1, wu

            x0, x1, wx = axis(cx, W)
            y0, y1, wy = axis(cy, H)
            z0, z1, wz = axis(cz, D)
            wx0 = 1.0 - wx
            wy0 = 1.0 - wy
            wz0 = 1.0 - wz
            k = 0
            for dz in (0, 1):
                zi = z1 if dz else z0
                wzs = wz if dz else wz0
                for dy in (0, 1):
                    yi = y1 if dy else y0
                    wys = wy if dy else wy0
                    zy = (zi * H + yi) * W
                    wzy = wzs * wys
                    for dx in (0, 1):
                        xi = x1 if dx else x0
                        wxs = wx if dx else wx0
                        idx_v[pl.ds(k * B + b0, 16)] = zy + xi
                        w_v[pl.ds(k * B + b0, 16)] = wzy * wxs
                        k += 1
            return c2

        lax.fori_loop(0, G, grp_body, 0, unroll=False)

        handles = [
            pltpu.async_copy(
                table_hbm.at[idx_v.at[pl.ds(j * ILEN, ILEN)]],
                rows_v.at[pl.ds(j * ILEN, ILEN)],
                sem,
            )
            for j in range(NSTREAM)
        ]
        for h in handles:
            h.wait()

        def acc_body(g, c2):
            b0 = g * 16
            wks = [w_v[pl.ds(k * B + b0, 16)] for k in range(8)]
            for j in range(16):
                lane_j = jnp.full((16,), j, jnp.int32)
                acc = None
                for k in range(8):
                    row = rows_v[k * B + b0 + j]
                    wjk = jnp.take(wks[k], lane_j)
                    term = row * wjk
                    acc = term if acc is None else acc + term
                out_v[b0 + j] = acc
            return c2

        lax.fori_loop(0, G, acc_body, 0, unroll=False)
        pltpu.sync_copy(out_v, out_hbm.at[pl.ds(base, B)])
        return carry

    lax.fori_loop(0, NBLK, blk_body, 0, unroll=False)


@jax.jit
def kernel(coords, features):
    ft = jnp.transpose(features[0], (1, 2, 3, 0)).reshape(V, C)
    xs = coords[:, 0]
    ys = coords[:, 1]
    zs = coords[:, 2]
    out = _sample_sc(xs, ys, zs, ft)
    return out[:, None, :]
